# Initial kernel scaffold; baseline (speedup 1.0000x reference)
import jax
import jax.numpy as jnp
from jax.experimental import pallas as pl


def kernel(node_feats, edge_feats, params, edge_index):
    raise NotImplementedError("write your pallas kernel here")


# speedup vs baseline: 1.6912x; 1.6912x over previous
import functools

import jax
import jax.numpy as jnp
from jax import lax
from jax.experimental import pallas as pl
from jax.experimental.pallas import tpu as pltpu
from jax.experimental.pallas import tpu_sc as plsc

N_NODES = 10000
N_EDGES = 160000
NP = 10240
EPS = 1e-5

NC = 2
NS = 16
NW = NC * NS
EPW = N_EDGES // NW
CHUNK = 1000
NSLICE = NP // NS

def _mesh():
    return plsc.VectorSubcoreMesh(core_axis_name="c", subcore_axis_name="s")


def _sc_gather16_deg(node_feats, src, dst, ones, zeros16):
    @functools.partial(
        pl.kernel, mesh=_mesh(),
        compiler_params=pltpu.CompilerParams(use_tc_tiling_on_sc=False),
        out_type=[
            jax.ShapeDtypeStruct((N_EDGES, 16), jnp.float32),
            jax.ShapeDtypeStruct((N_EDGES, 16), jnp.float32),
            jax.ShapeDtypeStruct((NC, NP, 16), jnp.float32),
            jax.ShapeDtypeStruct((NC, NP, 16), jnp.float32),
        ],
        scratch_types=[
            pltpu.VMEM((CHUNK,), jnp.int32),
            pltpu.VMEM((CHUNK,), jnp.int32),
            pltpu.VMEM((CHUNK, 16), jnp.float32),
            pltpu.VMEM((CHUNK, 16), jnp.float32),
            pltpu.VMEM((CHUNK, 16), jnp.float32),
            pltpu.VMEM_SHARED((NP, 16), jnp.float32),
            pltpu.VMEM_SHARED((NP, 16), jnp.float32),
            pltpu.SemaphoreType.DMA,
        ],
    )
    def k(nf_hbm, src_hbm, dst_hbm, ones_hbm, zero_hbm,
          g1d_hbm, g1s_hbm, indeg_hbm, outdeg_hbm,
          idx_d, idx_s, rows_d, rows_s, ones_v, sh_in, sh_out, sem):
        cid = lax.axis_index("c")
        sid = lax.axis_index("s")
        wid = sid * NC + cid
        base = wid * EPW
        pltpu.sync_copy(zero_hbm.at[pl.ds(sid * NSLICE, NSLICE)],
                        sh_in.at[pl.ds(sid * NSLICE, NSLICE)])
        pltpu.sync_copy(zero_hbm.at[pl.ds(sid * NSLICE, NSLICE)],
                        sh_out.at[pl.ds(sid * NSLICE, NSLICE)])
        pltpu.sync_copy(ones_hbm, ones_v)
        plsc.subcore_barrier()

        def step(ci, _):
            off = base + ci * CHUNK
            pltpu.sync_copy(dst_hbm.at[pl.ds(off, CHUNK)], idx_d)
            pltpu.sync_copy(src_hbm.at[pl.ds(off, CHUNK)], idx_s)
            pltpu.async_copy(nf_hbm.at[idx_d], rows_d, sem).wait()
            pltpu.async_copy(nf_hbm.at[idx_s], rows_s, sem).wait()
            pltpu.sync_copy(rows_d, g1d_hbm.at[pl.ds(off, CHUNK)])
            pltpu.sync_copy(rows_s, g1s_hbm.at[pl.ds(off, CHUNK)])
            pltpu.sync_copy(ones_v, sh_in.at[idx_d], add=True)
            pltpu.sync_copy(ones_v, sh_out.at[idx_s], add=True)
            return 0

        lax.fori_loop(0, EPW // CHUNK, step, 0)
        plsc.subcore_barrier()
        pltpu.sync_copy(sh_in.at[pl.ds(sid * NSLICE, NSLICE)],
                        indeg_hbm.at[cid, pl.ds(sid * NSLICE, NSLICE)])
        pltpu.sync_copy(sh_out.at[pl.ds(sid * NSLICE, NSLICE)],
                        outdeg_hbm.at[cid, pl.ds(sid * NSLICE, NSLICE)])

    return k(node_feats, src, dst, ones, zeros16)


def _sc_gather2(table, src, dst, feat):
    chunk = CHUNK if feat <= 64 else 200

    @functools.partial(
        pl.kernel, mesh=_mesh(),
        compiler_params=pltpu.CompilerParams(use_tc_tiling_on_sc=False),
        out_type=[
            jax.ShapeDtypeStruct((N_EDGES, feat), jnp.float32),
            jax.ShapeDtypeStruct((N_EDGES, feat), jnp.float32),
        ],
        scratch_types=[
            pltpu.VMEM((chunk,), jnp.int32),
            pltpu.VMEM((chunk, feat), jnp.float32),
            pltpu.SemaphoreType.DMA,
        ],
    )
    def k(tab_hbm, src_hbm, dst_hbm, gd_hbm, gs_hbm, idx_v, rows_v, sem):
        cid = lax.axis_index("c")
        sid = lax.axis_index("s")
        wid = sid * NC + cid
        base = wid * EPW
        for idx_hbm, out_hbm in ((dst_hbm, gd_hbm), (src_hbm, gs_hbm)):

            def step(ci, _, idx_hbm=idx_hbm, out_hbm=out_hbm):
                off = base + ci * chunk
                pltpu.sync_copy(idx_hbm.at[pl.ds(off, chunk)], idx_v)
                pltpu.async_copy(tab_hbm.at[idx_v], rows_v, sem).wait()
                pltpu.sync_copy(rows_v, out_hbm.at[pl.ds(off, chunk)])
                return 0

            lax.fori_loop(0, EPW // chunk, step, 0)

    return k(table, src, dst)


def _sc_scatter_add(h, dst, zeros, feat):
    chunk = CHUNK if feat <= 64 else 200

    @functools.partial(
        pl.kernel, mesh=_mesh(),
        compiler_params=pltpu.CompilerParams(use_tc_tiling_on_sc=False),
        out_type=[jax.ShapeDtypeStruct((NC, NP, feat), jnp.float32)],
        scratch_types=[
            pltpu.VMEM((chunk,), jnp.int32),
            pltpu.VMEM((chunk, feat), jnp.float32),
            pltpu.VMEM_SHARED((NP, feat), jnp.float32),
            pltpu.SemaphoreType.DMA,
        ],
    )
    def k(h_hbm, dst_hbm, zero_hbm, out_hbm, idx_v, rows_v, sh, sem):
        cid = lax.axis_index("c")
        sid = lax.axis_index("s")
        wid = sid * NC + cid
        base = wid * EPW
        pltpu.sync_copy(zero_hbm.at[pl.ds(sid * NSLICE, NSLICE)],
                        sh.at[pl.ds(sid * NSLICE, NSLICE)])
        plsc.subcore_barrier()

        def step(ci, _):
            off = base + ci * chunk
            pltpu.sync_copy(dst_hbm.at[pl.ds(off, chunk)], idx_v)
            pltpu.sync_copy(h_hbm.at[pl.ds(off, chunk)], rows_v)
            pltpu.sync_copy(rows_v, sh.at[idx_v], add=True)
            return 0

        lax.fori_loop(0, EPW // chunk, step, 0)
        plsc.subcore_barrier()
        pltpu.sync_copy(sh.at[pl.ds(sid * NSLICE, NSLICE)],
                        out_hbm.at[cid, pl.ds(sid * NSLICE, NSLICE)])

    return k(h, dst, zeros)[0]


_BLK = 2000


def _fused_linear(xs, combos, bias, relu, want_stats, block=_BLK):
    rows = xs[0].shape[0]
    fo = combos[0][1].shape[1]
    n = len(xs)
    nc = len(combos)
    affs = [aff for (_, _, aff) in combos if aff is not None]
    grid = (rows // block,)

    def body(*refs):
        xrefs = refs[:n]
        wrefs = refs[n:n + nc]
        bref = refs[n + nc]
        arefs = refs[n + nc + 1:n + nc + 1 + len(affs)]
        yref = refs[n + nc + 1 + len(affs)]
        vals = [xr[...] for xr in xrefs]
        acc = None
        ai = 0
        for (spec, _, aff), wr in zip(combos, wrefs):
            v = vals[spec] if isinstance(spec, int) else \
                vals[spec[0]] - vals[spec[1]]
            if aff is not None:
                ar = arefs[ai]
                ai += 1
                v = v * ar[0:1, :] + ar[1:2, :]
            t = jnp.dot(v, wr[...], preferred_element_type=jnp.float32)
            acc = t if acc is None else acc + t
        acc = acc + bref[0:1, :]
        if relu:
            acc = jnp.maximum(acc, 0.0)
        yref[...] = acc
        if want_stats:
            sref, qref = refs[n + nc + 2 + len(affs):n + nc + 4 + len(affs)]

            @pl.when(pl.program_id(0) == 0)
            def _():
                sref[...] = jnp.zeros_like(sref)
                qref[...] = jnp.zeros_like(qref)

            sref[...] += acc.reshape(block // 8, 8, fo).sum(0)
            qref[...] += (acc * acc).reshape(block // 8, 8, fo).sum(0)

    in_specs = (
        [pl.BlockSpec((block, x.shape[1]), lambda i: (i, 0)) for x in xs]
        + [pl.BlockSpec(w.shape, lambda i: (0, 0)) for (_, w, _) in combos]
        + [pl.BlockSpec((8, fo), lambda i: (0, 0))]
        + [pl.BlockSpec((8, a.shape[0]), lambda i: (0, 0)) for (a, _) in affs]
    )
    out_shape = [jax.ShapeDtypeStruct((rows, fo), jnp.float32)]
    out_specs = [pl.BlockSpec((block, fo), lambda i: (i, 0))]
    if want_stats:
        out_shape += [jax.ShapeDtypeStruct((8, fo), jnp.float32)] * 2
        out_specs += [pl.BlockSpec((8, fo), lambda i: (0, 0))] * 2
    bias8 = jnp.broadcast_to(bias[None, :], (8, fo))
    acs = [jnp.concatenate([a[None, :], c[None, :],
                            jnp.zeros((6, a.shape[0]), jnp.float32)], axis=0)
           for (a, c) in affs]
    res = pl.pallas_call(body, grid=grid, in_specs=in_specs,
                         out_specs=out_specs, out_shape=out_shape)(
                             *xs, *[w for (_, w, _) in combos], bias8, *acs)
    if want_stats:
        y, s, q = res
        return y, s.sum(0), q.sum(0)
    return res[0]


def _head_mlp(x, ws, bs, relus, rows, aff=None, block=_BLK):
    nl = len(ws)
    grid = (rows // block,)
    fo = ws[-1].shape[1]

    def body(xref, *refs):
        wrefs = refs[:nl]
        brefs = refs[nl:2 * nl]
        if aff is not None:
            aref = refs[2 * nl]
            yref = refs[2 * nl + 1]
        else:
            yref = refs[2 * nl]
        h = xref[...]
        if aff is not None:
            h = h * aref[0:1, :] + aref[1:2, :]
        for k in range(nl):
            h = jnp.dot(h, wrefs[k][...], preferred_element_type=jnp.float32)
            h = h + brefs[k][0:1, :]
            if relus[k]:
                h = jnp.maximum(h, 0.0)
        yref[...] = h

    in_specs = (
        [pl.BlockSpec((block, x.shape[1]), lambda i: (i, 0))]
        + [pl.BlockSpec(w.shape, lambda i: (0, 0)) for w in ws]
        + [pl.BlockSpec((8, w.shape[1]), lambda i: (0, 0)) for w in ws]
    )
    b8 = [jnp.broadcast_to(b[None, :], (8, b.shape[0])) for b in bs]
    args = [x, *ws, *b8]
    if aff is not None:
        a, c = aff
        in_specs.append(pl.BlockSpec((8, a.shape[0]), lambda i: (0, 0)))
        args.append(jnp.concatenate(
            [a[None, :], c[None, :],
             jnp.zeros((6, a.shape[0]), jnp.float32)], axis=0))
    return pl.pallas_call(
        body, grid=grid, in_specs=in_specs,
        out_specs=pl.BlockSpec((block, fo), lambda i: (i, 0)),
        out_shape=jax.ShapeDtypeStruct((rows, fo), jnp.float32),
    )(*args)


def _stats_only(xs, cols_spec, block=_BLK):
    rows = xs[0].shape[0]
    n = len(xs)
    grid = (rows // block,)

    def body(*refs):
        xrefs = refs[:n]
        orefs = refs[n:]

        @pl.when(pl.program_id(0) == 0)
        def _():
            for r in orefs:
                r[...] = jnp.zeros_like(r)

        vals = [xr[...] for xr in xrefs]
        cols = [vals[c] if isinstance(c, int) else vals[c[0]] - vals[c[1]]
                for c in cols_spec]
        for k, v in enumerate(cols):
            f = v.shape[1]
            orefs[2 * k][...] += v.reshape(block // 8, 8, f).sum(0)
            orefs[2 * k + 1][...] += (v * v).reshape(block // 8, 8, f).sum(0)

    in_specs = [pl.BlockSpec((block, x.shape[1]), lambda i: (i, 0)) for x in xs]
    feats = [xs[c].shape[1] if isinstance(c, int) else xs[c[0]].shape[1]
             for c in cols_spec]
    out_shape = []
    out_specs = []
    for f in feats:
        out_shape += [jax.ShapeDtypeStruct((8, f), jnp.float32)] * 2
        out_specs += [pl.BlockSpec((8, f), lambda i: (0, 0))] * 2
    res = pl.pallas_call(body, grid=grid, in_specs=in_specs,
                         out_specs=out_specs, out_shape=out_shape)(*xs)
    return [r.sum(0) for r in res]


_NBLK = 2048


def _node_affine(P, indeg_p, outdeg_p, a, c, feat):
    grid = (NP // _NBLK,)

    def body(pref, iref, oref, acref, xref, ss, qs, sd, qd):
        s = pref[0] + pref[1]
        ind = iref[0, :, 0:1] + iref[1, :, 0:1]
        outd = oref[0, :, 0:1] + oref[1, :, 0:1]
        x = acref[0:1, :] * s + acref[4:5, :] * ind
        xref[...] = x

        @pl.when(pl.program_id(0) == 0)
        def _():
            for r in (ss, qs, sd, qd):
                r[...] = jnp.zeros_like(r)

        xx = x * x
        ss[...] += (x * outd).reshape(_NBLK // 8, 8, feat).sum(0)
        qs[...] += (xx * outd).reshape(_NBLK // 8, 8, feat).sum(0)
        sd[...] += (x * ind).reshape(_NBLK // 8, 8, feat).sum(0)
        qd[...] += (xx * ind).reshape(_NBLK // 8, 8, feat).sum(0)

    ac = jnp.concatenate([jnp.broadcast_to(a[None, :], (4, feat)),
                          jnp.broadcast_to(c[None, :], (4, feat))], axis=0)
    in_specs = [
        pl.BlockSpec((2, _NBLK, feat), lambda i: (0, i, 0)),
        pl.BlockSpec((2, _NBLK, 16), lambda i: (0, i, 0)),
        pl.BlockSpec((2, _NBLK, 16), lambda i: (0, i, 0)),
        pl.BlockSpec((8, feat), lambda i: (0, 0)),
    ]
    out_shape = [jax.ShapeDtypeStruct((NP, feat), jnp.float32)] + \
        [jax.ShapeDtypeStruct((8, feat), jnp.float32)] * 4
    out_specs = [pl.BlockSpec((_NBLK, feat), lambda i: (i, 0))] + \
        [pl.BlockSpec((8, feat), lambda i: (0, 0))] * 4
    res = pl.pallas_call(body, grid=grid, in_specs=in_specs,
                         out_specs=out_specs, out_shape=out_shape)(
                             P, indeg_p, outdeg_p, ac)
    x = res[0]
    return x, [r.sum(0) for r in res[1:]]


def _mv(s, q, n):
    mu = s / n
    return mu, q / n - mu * mu


def _aff(mu, var):
    a = 1.0 / jnp.sqrt(var + EPS)
    return a, -mu * a


def kernel(node_feats, edge_feats, params, edge_index):
    p = params
    E = N_EDGES
    src = edge_index[0]
    dst = edge_index[1]
    ones16 = jnp.ones((CHUNK, 16), jnp.float32)
    zeros16 = jnp.zeros((NP, 16), jnp.float32)

    g1d, g1s, indeg_p, outdeg_p = _sc_gather16_deg(
        node_feats, src, dst, ones16, zeros16)

    (s_a, q_a, s_b, q_b, s_e, q_e) = _stats_only(
        [g1d, g1s, edge_feats], [0, (1, 0), 2])
    aff_a = _aff(*_mv(s_a, q_a, E))
    aff_b = _aff(*_mv(s_b, q_b, E))
    aff_ef = _aff(*_mv(s_e, q_e, E))
    w0t = p["nmm1_w0"].T
    h, s, q = _fused_linear(
        [g1d, g1s],
        [(0, w0t[:16], aff_a), ((1, 0), w0t[16:], aff_b)],
        p["nmm1_b0"], True, True)
    h, s, q = _fused_linear(
        [h], [(0, p["nmm1_w1"].T, _aff(*_mv(s, q, E)))],
        p["nmm1_b1"], True, True)
    h13, s, q = _fused_linear(
        [h], [(0, p["nmm1_w2"].T, _aff(*_mv(s, q, E)))],
        p["nmm1_b2"], True, True)
    a3, c3 = _aff(*_mv(s, q, E))

    zeros64 = jnp.zeros((NP, 64), jnp.float32)
    S1 = _sc_scatter_add(h13, dst, zeros64, 64)
    x1, (ss1, qs1, sd1, qd1) = _node_affine(
        S1, indeg_p, outdeg_p, a3, c3, 64)

    g2d, g2s = _sc_gather2(x1, src, dst, 64)
    aff_xs1 = _aff(*_mv(ss1, qs1, E))
    aff_xd1 = _aff(*_mv(sd1, qd1, E))
    v0t = p["emm1_w0"].T
    h, s, q = _fused_linear(
        [edge_feats, g2s, g2d],
        [(0, v0t[:19], aff_ef), (1, v0t[19:83], aff_xs1),
         (2, v0t[83:147], aff_xd1)],
        p["emm1_b0"], True, True)
    h, s, q = _fused_linear(
        [h], [(0, p["emm1_w1"].T, _aff(*_mv(s, q, E)))],
        p["emm1_b1"], True, True)
    e1raw, s, q = _fused_linear(
        [h], [(0, p["emm1_w2"].T, _aff(*_mv(s, q, E)))],
        p["emm1_b2"], True, True)
    aff_e1 = _aff(*_mv(s, q, E))

    w0t3 = p["nmm2_w0"].T
    h, s, q = _fused_linear(
        [g2d, g2s],
        [(0, w0t3[:64], None), ((1, 0), w0t3[64:], None)],
        p["nmm2_b0"], True, True)
    h, s, q = _fused_linear(
        [h], [(0, p["nmm2_w1"].T, _aff(*_mv(s, q, E)))],
        p["nmm2_b1"], True, True)
    h33, s, q = _fused_linear(
        [h], [(0, p["nmm2_w2"].T, _aff(*_mv(s, q, E)))],
        p["nmm2_b2"], True, True)
    a3c, c3c = _aff(*_mv(s, q, E))

    zeros128 = jnp.zeros((NP, 128), jnp.float32)
    S2 = _sc_scatter_add(h33, dst, zeros128, 128)
    x2, (ss2, qs2, sd2, qd2) = _node_affine(
        S2, indeg_p, outdeg_p, a3c, c3c, 128)

    g3d, g3s = _sc_gather2(x2, src, dst, 128)
    aff_xs2 = _aff(*_mv(ss2, qs2, E))
    aff_xd2 = _aff(*_mv(sd2, qd2, E))
    u0t = p["emm2_w0"].T
    h, s, q = _fused_linear(
        [e1raw, g3s, g3d],
        [(0, u0t[:76], aff_e1), (1, u0t[76:204], aff_xs2),
         (2, u0t[204:332], aff_xd2)],
        p["emm2_b0"], True, True)
    h, s, q = _fused_linear(
        [h], [(0, p["emm2_w1"].T, _aff(*_mv(s, q, E)))],
        p["emm2_b1"], True, True)
    e2raw, s, q = _fused_linear(
        [h], [(0, p["emm2_w2"].T, _aff(*_mv(s, q, E)))],
        p["emm2_b2"], True, True)
    aff_e2 = _aff(*_mv(s, q, E))

    nout = _head_mlp(
        x2,
        [p["nhead_w0"].T, p["nhead_w1"].T, p["nhead_w2"].T, p["nhead_w3"].T],
        [p["nhead_b0"], p["nhead_b1"], p["nhead_b2"], p["nhead_b3"]],
        [True, True, False, False], N_NODES)

    eout = _head_mlp(
        e2raw,
        [p["ehead_w0"].T, p["ehead_w1"].T, p["ehead_w2"].T, p["ehead_w3"].T,
         p["ehead_w4"].T],
        [p["ehead_b0"], p["ehead_b1"], p["ehead_b2"], p["ehead_b3"],
         p["ehead_b4"]],
        [False, True, True, False, False], E, aff=aff_e2)

    return (nout, eout)


# speedup vs baseline: 1.9614x; 1.1598x over previous
import functools

import jax
import jax.numpy as jnp
from jax import lax
from jax.experimental import pallas as pl
from jax.experimental.pallas import tpu as pltpu
from jax.experimental.pallas import tpu_sc as plsc

N_NODES = 10000
N_EDGES = 160000
NP = 10240
EPS = 1e-5

NC = 2
NS = 16
NW = NC * NS
EPW = N_EDGES // NW
CHUNK = 1000
NSLICE = NP // NS

def _mesh():
    return plsc.VectorSubcoreMesh(core_axis_name="c", subcore_axis_name="s")


def _sc_gather16_deg(node_feats, src, dst, ones, zeros16):
    @functools.partial(
        pl.kernel, mesh=_mesh(),
        compiler_params=pltpu.CompilerParams(use_tc_tiling_on_sc=False),
        out_type=[
            jax.ShapeDtypeStruct((N_EDGES, 16), jnp.float32),
            jax.ShapeDtypeStruct((N_EDGES, 16), jnp.float32),
            jax.ShapeDtypeStruct((NC, NP, 16), jnp.float32),
            jax.ShapeDtypeStruct((NC, NP, 16), jnp.float32),
        ],
        scratch_types=[
            pltpu.VMEM((CHUNK,), jnp.int32),
            pltpu.VMEM((CHUNK,), jnp.int32),
            pltpu.VMEM((CHUNK, 16), jnp.float32),
            pltpu.VMEM((CHUNK, 16), jnp.float32),
            pltpu.VMEM((CHUNK, 16), jnp.float32),
            pltpu.VMEM_SHARED((NP, 16), jnp.float32),
            pltpu.VMEM_SHARED((NP, 16), jnp.float32),
            pltpu.SemaphoreType.DMA,
            pltpu.SemaphoreType.DMA,
            pltpu.SemaphoreType.DMA,
            pltpu.SemaphoreType.DMA,
        ],
    )
    def k(nf_hbm, src_hbm, dst_hbm, ones_hbm, zero_hbm,
          g1d_hbm, g1s_hbm, indeg_hbm, outdeg_hbm,
          idx_d, idx_s, rows_d, rows_s, ones_v, sh_in, sh_out,
          sem_d, sem_s, sem_wd, sem_ws):
        cid = lax.axis_index("c")
        sid = lax.axis_index("s")
        wid = sid * NC + cid
        base = wid * EPW
        pltpu.sync_copy(zero_hbm.at[pl.ds(sid * NSLICE, NSLICE)],
                        sh_in.at[pl.ds(sid * NSLICE, NSLICE)])
        pltpu.sync_copy(zero_hbm.at[pl.ds(sid * NSLICE, NSLICE)],
                        sh_out.at[pl.ds(sid * NSLICE, NSLICE)])
        pltpu.sync_copy(ones_hbm, ones_v)
        plsc.subcore_barrier()

        def step(ci, _):
            off = base + ci * CHUNK
            id_ld = pltpu.async_copy(dst_hbm.at[pl.ds(off, CHUNK)], idx_d,
                                     sem_wd)
            is_ld = pltpu.async_copy(src_hbm.at[pl.ds(off, CHUNK)], idx_s,
                                     sem_ws)
            id_ld.wait()
            is_ld.wait()
            cd = pltpu.async_copy(nf_hbm.at[idx_d], rows_d, sem_d)
            cs = pltpu.async_copy(nf_hbm.at[idx_s], rows_s, sem_s)
            cd.wait()
            wd = pltpu.async_copy(rows_d, g1d_hbm.at[pl.ds(off, CHUNK)],
                                  sem_wd)
            cs.wait()
            ws = pltpu.async_copy(rows_s, g1s_hbm.at[pl.ds(off, CHUNK)],
                                  sem_ws)
            pltpu.sync_copy(ones_v, sh_in.at[idx_d], add=True)
            pltpu.sync_copy(ones_v, sh_out.at[idx_s], add=True)
            wd.wait()
            ws.wait()
            return 0

        lax.fori_loop(0, EPW // CHUNK, step, 0)
        plsc.subcore_barrier()
        pltpu.sync_copy(sh_in.at[pl.ds(sid * NSLICE, NSLICE)],
                        indeg_hbm.at[cid, pl.ds(sid * NSLICE, NSLICE)])
        pltpu.sync_copy(sh_out.at[pl.ds(sid * NSLICE, NSLICE)],
                        outdeg_hbm.at[cid, pl.ds(sid * NSLICE, NSLICE)])

    return k(node_feats, src, dst, ones, zeros16)


def _sc_gather2(table, src, dst, feat):
    chunk = 200
    nbuf = 4
    ngrp = EPW // (chunk * nbuf)
    tail = (EPW - ngrp * chunk * nbuf) // chunk

    @functools.partial(
        pl.kernel, mesh=_mesh(),
        compiler_params=pltpu.CompilerParams(use_tc_tiling_on_sc=False),
        out_type=[
            jax.ShapeDtypeStruct((N_EDGES, feat), jnp.float32),
            jax.ShapeDtypeStruct((N_EDGES, feat), jnp.float32),
        ],
        scratch_types=[
            pltpu.VMEM((EPW,), jnp.int32),
            pltpu.VMEM((chunk, feat), jnp.float32),
            pltpu.VMEM((chunk, feat), jnp.float32),
            pltpu.VMEM((chunk, feat), jnp.float32),
            pltpu.VMEM((chunk, feat), jnp.float32),
            pltpu.SemaphoreType.DMA,
            pltpu.SemaphoreType.DMA,
            pltpu.SemaphoreType.DMA,
            pltpu.SemaphoreType.DMA,
            pltpu.SemaphoreType.DMA,
        ],
    )
    def k(tab_hbm, src_hbm, dst_hbm, gd_hbm, gs_hbm,
          idx_all, r0, r1, r2, r3, s0, s1, s2, s3, sw):
        cid = lax.axis_index("c")
        sid = lax.axis_index("s")
        wid = sid * NC + cid
        base = wid * EPW
        rows = (r0, r1, r2, r3)
        sems = (s0, s1, s2, s3)
        for idx_hbm, out_hbm in ((dst_hbm, gd_hbm), (src_hbm, gs_hbm)):
            pltpu.sync_copy(idx_hbm.at[pl.ds(base, EPW)], idx_all)

            def grp(g, _, out_hbm=out_hbm):
                loc = g * chunk * nbuf
                cps = []
                for b in range(nbuf):
                    iv = idx_all.at[pl.ds(loc + b * chunk, chunk)]
                    cps.append(pltpu.async_copy(tab_hbm.at[iv], rows[b],
                                                sems[b]))
                for b in range(nbuf):
                    cps[b].wait()
                    pltpu.async_copy(
                        rows[b],
                        out_hbm.at[pl.ds(base + loc + b * chunk, chunk)],
                        sems[b]).wait()
                return 0

            lax.fori_loop(0, ngrp, grp, 0)
            for t in range(tail):
                loc = (ngrp * nbuf + t) * chunk
                iv = idx_all.at[pl.ds(loc, chunk)]
                pltpu.async_copy(tab_hbm.at[iv], rows[t], sems[t]).wait()
                pltpu.sync_copy(rows[t],
                                out_hbm.at[pl.ds(base + loc, chunk)])

    return k(table, src, dst)


def _sc_scatter_add(h, dst, zeros, feat):
    chunk = 200 if feat <= 64 else 40
    nbuf = 4 if feat <= 64 else 8
    ngrp = EPW // (chunk * nbuf)
    tail = (EPW - ngrp * chunk * nbuf) // chunk

    @functools.partial(
        pl.kernel, mesh=_mesh(),
        compiler_params=pltpu.CompilerParams(use_tc_tiling_on_sc=False),
        out_type=[jax.ShapeDtypeStruct((NC, NP, feat), jnp.float32)],
        scratch_types=(
            [pltpu.VMEM((chunk,), jnp.int32)] * nbuf
            + [pltpu.VMEM((chunk, feat), jnp.float32)] * nbuf
            + [pltpu.VMEM_SHARED((NP, feat), jnp.float32)]
            + [pltpu.SemaphoreType.DMA] * (nbuf + 1)
        ),
    )
    def k(h_hbm, dst_hbm, zero_hbm, out_hbm, *scr):
        idxs = scr[:nbuf]
        rows = scr[nbuf:2 * nbuf]
        sh = scr[2 * nbuf]
        sems = scr[2 * nbuf + 1:3 * nbuf + 1]
        ss = scr[3 * nbuf + 1]
        cid = lax.axis_index("c")
        sid = lax.axis_index("s")
        wid = sid * NC + cid
        base = wid * EPW
        pltpu.sync_copy(zero_hbm.at[pl.ds(sid * NSLICE, NSLICE)],
                        sh.at[pl.ds(sid * NSLICE, NSLICE)])
        plsc.subcore_barrier()

        def grp(g, _):
            off = base + g * chunk * nbuf
            cps = []
            for b in range(nbuf):
                o = off + b * chunk
                ci = pltpu.async_copy(dst_hbm.at[pl.ds(o, chunk)], idxs[b],
                                      sems[b])
                cr = pltpu.async_copy(h_hbm.at[pl.ds(o, chunk)], rows[b],
                                      ss)
                cps.append((ci, cr))
            for b in range(nbuf):
                cps[b][0].wait()
                cps[b][1].wait()
                pltpu.sync_copy(rows[b], sh.at[idxs[b]], add=True)
            return 0

        lax.fori_loop(0, ngrp, grp, 0)
        for t in range(tail):
            o = base + (ngrp * nbuf + t) * chunk
            pltpu.async_copy(dst_hbm.at[pl.ds(o, chunk)], idxs[t],
                             sems[t]).wait()
            pltpu.async_copy(h_hbm.at[pl.ds(o, chunk)], rows[t], ss).wait()
            pltpu.sync_copy(rows[t], sh.at[idxs[t]], add=True)
        plsc.subcore_barrier()
        pltpu.sync_copy(sh.at[pl.ds(sid * NSLICE, NSLICE)],
                        out_hbm.at[cid, pl.ds(sid * NSLICE, NSLICE)])

    return k(h, dst, zeros)[0]


_BLK = 4000


def _bn_from_sums(sref, qref):
    s = jnp.sum(sref[...], axis=0, keepdims=True)
    q = jnp.sum(qref[...], axis=0, keepdims=True)
    mu = s * (1.0 / N_EDGES)
    var = q * (1.0 / N_EDGES) - mu * mu
    a = jax.lax.rsqrt(var + EPS)
    return a, -mu * a


def _fused_linear(xs, combos, bias, relu, want_stats, block=_BLK):
    rows = xs[0].shape[0]
    fo = combos[0][1].shape[1]
    n = len(xs)
    nc = len(combos)
    affs = [aff for (_, _, aff) in combos if aff is not None]
    grid = (rows // block,)

    def body(*refs):
        xrefs = refs[:n]
        wrefs = refs[n:n + nc]
        bref = refs[n + nc]
        arefs = refs[n + nc + 1:n + nc + 1 + 2 * len(affs)]
        yref = refs[n + nc + 1 + 2 * len(affs)]
        vals = [xr[...] for xr in xrefs]
        acc = None
        ai = 0
        for (spec, _, aff), wr in zip(combos, wrefs):
            v = vals[spec] if isinstance(spec, int) else \
                vals[spec[0]] - vals[spec[1]]
            if aff is not None:
                a8, c8 = _bn_from_sums(arefs[2 * ai], arefs[2 * ai + 1])
                ai += 1
                v = v * a8 + c8
            t = jnp.dot(v, wr[...], preferred_element_type=jnp.float32)
            acc = t if acc is None else acc + t
        acc = acc + bref[0:1, :]
        if relu:
            acc = jnp.maximum(acc, 0.0)
        yref[...] = acc
        if want_stats:
            k0 = n + nc + 2 + 2 * len(affs)
            sref, qref = refs[k0:k0 + 2]

            @pl.when(pl.program_id(0) == 0)
            def _():
                sref[...] = jnp.zeros_like(sref)
                qref[...] = jnp.zeros_like(qref)

            sref[...] += acc.reshape(block // 8, 8, fo).sum(0)
            qref[...] += (acc * acc).reshape(block // 8, 8, fo).sum(0)

    in_specs = (
        [pl.BlockSpec((block, x.shape[1]), lambda i: (i, 0)) for x in xs]
        + [pl.BlockSpec(w.shape, lambda i: (0, 0)) for (_, w, _) in combos]
        + [pl.BlockSpec((8, fo), lambda i: (0, 0))]
    )
    for (s8, q8) in affs:
        in_specs += [pl.BlockSpec(s8.shape, lambda i: (0, 0))] * 2
    out_shape = [jax.ShapeDtypeStruct((rows, fo), jnp.float32)]
    out_specs = [pl.BlockSpec((block, fo), lambda i: (i, 0))]
    if want_stats:
        out_shape += [jax.ShapeDtypeStruct((8, fo), jnp.float32)] * 2
        out_specs += [pl.BlockSpec((8, fo), lambda i: (0, 0))] * 2
    bias8 = jnp.broadcast_to(bias[None, :], (8, fo))
    aflat = [r for (s8, q8) in affs for r in (s8, q8)]
    res = pl.pallas_call(body, grid=grid, in_specs=in_specs,
                         out_specs=out_specs, out_shape=out_shape)(
                             *xs, *[w for (_, w, _) in combos], bias8, *aflat)
    if want_stats:
        return res[0], res[1], res[2]
    return res[0]


def _head_mlp(x, ws, bs, relus, rows, aff=None, block=_BLK):
    nl = len(ws)
    grid = (rows // block,)
    fo = ws[-1].shape[1]

    def body(xref, *refs):
        wrefs = refs[:nl]
        brefs = refs[nl:2 * nl]
        if aff is not None:
            s8, q8 = refs[2 * nl:2 * nl + 2]
            yref = refs[2 * nl + 2]
        else:
            yref = refs[2 * nl]
        h = xref[...]
        if aff is not None:
            a8, c8 = _bn_from_sums(s8, q8)
            h = h * a8 + c8
        for k in range(nl):
            h = jnp.dot(h, wrefs[k][...], preferred_element_type=jnp.float32)
            h = h + brefs[k][0:1, :]
            if relus[k]:
                h = jnp.maximum(h, 0.0)
        yref[...] = h

    in_specs = (
        [pl.BlockSpec((block, x.shape[1]), lambda i: (i, 0))]
        + [pl.BlockSpec(w.shape, lambda i: (0, 0)) for w in ws]
        + [pl.BlockSpec((8, w.shape[1]), lambda i: (0, 0)) for w in ws]
    )
    b8 = [jnp.broadcast_to(b[None, :], (8, b.shape[0])) for b in bs]
    args = [x, *ws, *b8]
    if aff is not None:
        s8, q8 = aff
        in_specs += [pl.BlockSpec(s8.shape, lambda i: (0, 0))] * 2
        args += [s8, q8]
    return pl.pallas_call(
        body, grid=grid, in_specs=in_specs,
        out_specs=pl.BlockSpec((block, fo), lambda i: (i, 0)),
        out_shape=jax.ShapeDtypeStruct((rows, fo), jnp.float32),
    )(*args)


def _stats_only(xs, cols_spec, block=_BLK):
    rows = xs[0].shape[0]
    n = len(xs)
    grid = (rows // block,)

    def body(*refs):
        xrefs = refs[:n]
        orefs = refs[n:]

        @pl.when(pl.program_id(0) == 0)
        def _():
            for r in orefs:
                r[...] = jnp.zeros_like(r)

        vals = [xr[...] for xr in xrefs]
        cols = [vals[c] if isinstance(c, int) else vals[c[0]] - vals[c[1]]
                for c in cols_spec]
        for k, v in enumerate(cols):
            f = v.shape[1]
            orefs[2 * k][...] += v.reshape(block // 8, 8, f).sum(0)
            orefs[2 * k + 1][...] += (v * v).reshape(block // 8, 8, f).sum(0)

    in_specs = [pl.BlockSpec((block, x.shape[1]), lambda i: (i, 0)) for x in xs]
    feats = [xs[c].shape[1] if isinstance(c, int) else xs[c[0]].shape[1]
             for c in cols_spec]
    out_shape = []
    out_specs = []
    for f in feats:
        out_shape += [jax.ShapeDtypeStruct((8, f), jnp.float32)] * 2
        out_specs += [pl.BlockSpec((8, f), lambda i: (0, 0))] * 2
    res = pl.pallas_call(body, grid=grid, in_specs=in_specs,
                         out_specs=out_specs, out_shape=out_shape)(*xs)
    return list(res)


_NBLK = 2048


def _node_affine(P, indeg_p, outdeg_p, s8, q8, feat):
    grid = (NP // _NBLK,)

    def body(pref, iref, oref, sref, qref, xref, ss, qs, sd, qd):
        a8, c8 = _bn_from_sums(sref, qref)
        s = pref[0] + pref[1]
        ind = iref[0, :, 0:1] + iref[1, :, 0:1]
        outd = oref[0, :, 0:1] + oref[1, :, 0:1]
        x = a8 * s + c8 * ind
        xref[...] = x

        @pl.when(pl.program_id(0) == 0)
        def _():
            for r in (ss, qs, sd, qd):
                r[...] = jnp.zeros_like(r)

        xx = x * x
        ss[...] += (x * outd).reshape(_NBLK // 8, 8, feat).sum(0)
        qs[...] += (xx * outd).reshape(_NBLK // 8, 8, feat).sum(0)
        sd[...] += (x * ind).reshape(_NBLK // 8, 8, feat).sum(0)
        qd[...] += (xx * ind).reshape(_NBLK // 8, 8, feat).sum(0)

    in_specs = [
        pl.BlockSpec((2, _NBLK, feat), lambda i: (0, i, 0)),
        pl.BlockSpec((2, _NBLK, 16), lambda i: (0, i, 0)),
        pl.BlockSpec((2, _NBLK, 16), lambda i: (0, i, 0)),
        pl.BlockSpec((8, feat), lambda i: (0, 0)),
        pl.BlockSpec((8, feat), lambda i: (0, 0)),
    ]
    out_shape = [jax.ShapeDtypeStruct((NP, feat), jnp.float32)] + \
        [jax.ShapeDtypeStruct((8, feat), jnp.float32)] * 4
    out_specs = [pl.BlockSpec((_NBLK, feat), lambda i: (i, 0))] + \
        [pl.BlockSpec((8, feat), lambda i: (0, 0))] * 4
    res = pl.pallas_call(body, grid=grid, in_specs=in_specs,
                         out_specs=out_specs, out_shape=out_shape)(
                             P, indeg_p, outdeg_p, s8, q8)
    return res[0], res[1:]


def kernel(node_feats, edge_feats, params, edge_index):
    p = params
    E = N_EDGES
    src = edge_index[0]
    dst = edge_index[1]
    ones16 = jnp.ones((CHUNK, 16), jnp.float32)
    zeros16 = jnp.zeros((NP, 16), jnp.float32)

    g1d, g1s, indeg_p, outdeg_p = _sc_gather16_deg(
        node_feats, src, dst, ones16, zeros16)

    aff_ef = _stats_only([edge_feats], [0])
    (s_a, q_a, s_b, q_b) = _stats_only([g1d, g1s], [0, (1, 0)])
    w0t = p["nmm1_w0"].T
    h, s, q = _fused_linear(
        [g1d, g1s],
        [(0, w0t[:16], (s_a, q_a)), ((1, 0), w0t[16:], (s_b, q_b))],
        p["nmm1_b0"], True, True)
    h, s, q = _fused_linear(
        [h], [(0, p["nmm1_w1"].T, (s, q))], p["nmm1_b1"], True, True)
    h13, s, q = _fused_linear(
        [h], [(0, p["nmm1_w2"].T, (s, q))], p["nmm1_b2"], True, True)

    zeros64 = jnp.zeros((NP, 64), jnp.float32)
    S1 = _sc_scatter_add(h13, dst, zeros64, 64)
    x1, (ss1, qs1, sd1, qd1) = _node_affine(S1, indeg_p, outdeg_p, s, q, 64)

    g2d, g2s = _sc_gather2(x1, src, dst, 64)

    w0t3 = p["nmm2_w0"].T
    h, s, q = _fused_linear(
        [g2d, g2s],
        [(0, w0t3[:64], None), ((1, 0), w0t3[64:], None)],
        p["nmm2_b0"], True, True)
    h, s, q = _fused_linear(
        [h], [(0, p["nmm2_w1"].T, (s, q))], p["nmm2_b1"], True, True)
    h33, s3, q3 = _fused_linear(
        [h], [(0, p["nmm2_w2"].T, (s, q))], p["nmm2_b2"], True, True)

    zeros128 = jnp.zeros((NP, 128), jnp.float32)
    S2 = _sc_scatter_add(h33, dst, zeros128, 128)

    v0t = p["emm1_w0"].T
    h, s, q = _fused_linear(
        [edge_feats, g2s, g2d],
        [(0, v0t[:19], aff_ef), (1, v0t[19:83], (ss1, qs1)),
         (2, v0t[83:147], (sd1, qd1))],
        p["emm1_b0"], True, True)
    h, s, q = _fused_linear(
        [h], [(0, p["emm1_w1"].T, (s, q))], p["emm1_b1"], True, True)
    e1raw, s_e1, q_e1 = _fused_linear(
        [h], [(0, p["emm1_w2"].T, (s, q))], p["emm1_b2"], True, True)

    x2, (ss2, qs2, sd2, qd2) = _node_affine(S2, indeg_p, outdeg_p, s3, q3, 128)

    g3d, g3s = _sc_gather2(x2, src, dst, 128)
    u0t = p["emm2_w0"].T
    h, s, q = _fused_linear(
        [e1raw, g3s, g3d],
        [(0, u0t[:76], (s_e1, q_e1)), (1, u0t[76:204], (ss2, qs2)),
         (2, u0t[204:332], (sd2, qd2))],
        p["emm2_b0"], True, True)
    h, s, q = _fused_linear(
        [h], [(0, p["emm2_w1"].T, (s, q))], p["emm2_b1"], True, True)
    e2raw, s, q = _fused_linear(
        [h], [(0, p["emm2_w2"].T, (s, q))], p["emm2_b2"], True, True)
    aff_e2 = (s, q)

    nout = _head_mlp(
        x2,
        [p["nhead_w0"].T, p["nhead_w1"].T, p["nhead_w2"].T, p["nhead_w3"].T],
        [p["nhead_b0"], p["nhead_b1"], p["nhead_b2"], p["nhead_b3"]],
        [True, True, False, False], N_NODES, block=2000)

    eout = _head_mlp(
        e2raw,
        [p["ehead_w0"].T, p["ehead_w1"].T, p["ehead_w2"].T, p["ehead_w3"].T,
         p["ehead_w4"].T],
        [p["ehead_b0"], p["ehead_b1"], p["ehead_b2"], p["ehead_b3"],
         p["ehead_b4"]],
        [False, True, True, False, False], E, aff=aff_e2)

    return (nout, eout)


# speedup vs baseline: 2.0669x; 1.0538x over previous
import functools

import jax
import jax.numpy as jnp
from jax import lax
from jax.experimental import pallas as pl
from jax.experimental.pallas import tpu as pltpu
from jax.experimental.pallas import tpu_sc as plsc

N_NODES = 10000
N_EDGES = 160000
NP = 10240
EPS = 1e-5

NC = 2
NS = 16
NW = NC * NS
EPW = N_EDGES // NW
CHUNK = 1000
NSLICE = NP // NS

def _mesh():
    return plsc.VectorSubcoreMesh(core_axis_name="c", subcore_axis_name="s")


def _sc_gather16_deg(node_feats, src, dst, ones, zeros16):
    @functools.partial(
        pl.kernel, mesh=_mesh(),
        compiler_params=pltpu.CompilerParams(use_tc_tiling_on_sc=False),
        out_type=[
            jax.ShapeDtypeStruct((N_EDGES, 16), jnp.float32),
            jax.ShapeDtypeStruct((N_EDGES, 16), jnp.float32),
            jax.ShapeDtypeStruct((NC, NP, 16), jnp.float32),
            jax.ShapeDtypeStruct((NC, NP, 16), jnp.float32),
        ],
        scratch_types=[
            pltpu.VMEM((CHUNK,), jnp.int32),
            pltpu.VMEM((CHUNK,), jnp.int32),
            pltpu.VMEM((CHUNK, 16), jnp.float32),
            pltpu.VMEM((CHUNK, 16), jnp.float32),
            pltpu.VMEM((CHUNK, 16), jnp.float32),
            pltpu.VMEM_SHARED((NP, 16), jnp.float32),
            pltpu.VMEM_SHARED((NP, 16), jnp.float32),
            pltpu.SemaphoreType.DMA,
            pltpu.SemaphoreType.DMA,
            pltpu.SemaphoreType.DMA,
            pltpu.SemaphoreType.DMA,
        ],
    )
    def k(nf_hbm, src_hbm, dst_hbm, ones_hbm, zero_hbm,
          g1d_hbm, g1s_hbm, indeg_hbm, outdeg_hbm,
          idx_d, idx_s, rows_d, rows_s, ones_v, sh_in, sh_out,
          sem_d, sem_s, sem_wd, sem_ws):
        cid = lax.axis_index("c")
        sid = lax.axis_index("s")
        wid = sid * NC + cid
        base = wid * EPW
        pltpu.sync_copy(zero_hbm.at[pl.ds(sid * NSLICE, NSLICE)],
                        sh_in.at[pl.ds(sid * NSLICE, NSLICE)])
        pltpu.sync_copy(zero_hbm.at[pl.ds(sid * NSLICE, NSLICE)],
                        sh_out.at[pl.ds(sid * NSLICE, NSLICE)])
        pltpu.sync_copy(ones_hbm, ones_v)
        plsc.subcore_barrier()

        def step(ci, _):
            off = base + ci * CHUNK
            id_ld = pltpu.async_copy(dst_hbm.at[pl.ds(off, CHUNK)], idx_d,
                                     sem_wd)
            is_ld = pltpu.async_copy(src_hbm.at[pl.ds(off, CHUNK)], idx_s,
                                     sem_ws)
            id_ld.wait()
            is_ld.wait()
            cd = pltpu.async_copy(nf_hbm.at[idx_d], rows_d, sem_d)
            cs = pltpu.async_copy(nf_hbm.at[idx_s], rows_s, sem_s)
            cd.wait()
            wd = pltpu.async_copy(rows_d, g1d_hbm.at[pl.ds(off, CHUNK)],
                                  sem_wd)
            cs.wait()
            ws = pltpu.async_copy(rows_s, g1s_hbm.at[pl.ds(off, CHUNK)],
                                  sem_ws)
            pltpu.sync_copy(ones_v, sh_in.at[idx_d], add=True)
            pltpu.sync_copy(ones_v, sh_out.at[idx_s], add=True)
            wd.wait()
            ws.wait()
            return 0

        lax.fori_loop(0, EPW // CHUNK, step, 0)
        plsc.subcore_barrier()
        pltpu.sync_copy(sh_in.at[pl.ds(sid * NSLICE, NSLICE)],
                        indeg_hbm.at[cid, pl.ds(sid * NSLICE, NSLICE)])
        pltpu.sync_copy(sh_out.at[pl.ds(sid * NSLICE, NSLICE)],
                        outdeg_hbm.at[cid, pl.ds(sid * NSLICE, NSLICE)])

    return k(node_feats, src, dst, ones, zeros16)


def _sc_gather2(table, src, dst, feat):
    chunk = 200
    nbuf = 4
    ngrp = EPW // (chunk * nbuf)
    tail = (EPW - ngrp * chunk * nbuf) // chunk

    @functools.partial(
        pl.kernel, mesh=_mesh(),
        compiler_params=pltpu.CompilerParams(use_tc_tiling_on_sc=False),
        out_type=[
            jax.ShapeDtypeStruct((N_EDGES, feat), jnp.float32),
            jax.ShapeDtypeStruct((N_EDGES, feat), jnp.float32),
        ],
        scratch_types=[
            pltpu.VMEM((EPW,), jnp.int32),
            pltpu.VMEM((chunk, feat), jnp.float32),
            pltpu.VMEM((chunk, feat), jnp.float32),
            pltpu.VMEM((chunk, feat), jnp.float32),
            pltpu.VMEM((chunk, feat), jnp.float32),
            pltpu.SemaphoreType.DMA,
            pltpu.SemaphoreType.DMA,
            pltpu.SemaphoreType.DMA,
            pltpu.SemaphoreType.DMA,
            pltpu.SemaphoreType.DMA,
        ],
    )
    def k(tab_hbm, src_hbm, dst_hbm, gd_hbm, gs_hbm,
          idx_all, r0, r1, r2, r3, s0, s1, s2, s3, sw):
        cid = lax.axis_index("c")
        sid = lax.axis_index("s")
        wid = sid * NC + cid
        base = wid * EPW
        rows = (r0, r1, r2, r3)
        sems = (s0, s1, s2, s3)
        for idx_hbm, out_hbm in ((dst_hbm, gd_hbm), (src_hbm, gs_hbm)):
            pltpu.sync_copy(idx_hbm.at[pl.ds(base, EPW)], idx_all)

            def grp(g, _, out_hbm=out_hbm):
                loc = g * chunk * nbuf
                cps = []
                for b in range(nbuf):
                    iv = idx_all.at[pl.ds(loc + b * chunk, chunk)]
                    cps.append(pltpu.async_copy(tab_hbm.at[iv], rows[b],
                                                sems[b]))
                for b in range(nbuf):
                    cps[b].wait()
                    pltpu.async_copy(
                        rows[b],
                        out_hbm.at[pl.ds(base + loc + b * chunk, chunk)],
                        sems[b]).wait()
                return 0

            lax.fori_loop(0, ngrp, grp, 0)
            for t in range(tail):
                loc = (ngrp * nbuf + t) * chunk
                iv = idx_all.at[pl.ds(loc, chunk)]
                pltpu.async_copy(tab_hbm.at[iv], rows[t], sems[t]).wait()
                pltpu.sync_copy(rows[t],
                                out_hbm.at[pl.ds(base + loc, chunk)])

    return k(table, src, dst)


def _sc_scatter_add(h, dst, zeros, feat):
    chunk = 200 if feat <= 64 else 40
    nbuf = 4 if feat <= 64 else 8
    ngrp = EPW // (chunk * nbuf)
    tail = (EPW - ngrp * chunk * nbuf) // chunk

    @functools.partial(
        pl.kernel, mesh=_mesh(),
        compiler_params=pltpu.CompilerParams(use_tc_tiling_on_sc=False),
        out_type=[jax.ShapeDtypeStruct((NC, NP, feat), jnp.float32)],
        scratch_types=(
            [pltpu.VMEM((chunk,), jnp.int32)] * nbuf
            + [pltpu.VMEM((chunk, feat), jnp.float32)] * nbuf
            + [pltpu.VMEM_SHARED((NP, feat), jnp.float32)]
            + [pltpu.SemaphoreType.DMA] * (nbuf + 1)
        ),
    )
    def k(h_hbm, dst_hbm, zero_hbm, out_hbm, *scr):
        idxs = scr[:nbuf]
        rows = scr[nbuf:2 * nbuf]
        sh = scr[2 * nbuf]
        sems = scr[2 * nbuf + 1:3 * nbuf + 1]
        ss = scr[3 * nbuf + 1]
        cid = lax.axis_index("c")
        sid = lax.axis_index("s")
        wid = sid * NC + cid
        base = wid * EPW
        pltpu.sync_copy(zero_hbm.at[pl.ds(sid * NSLICE, NSLICE)],
                        sh.at[pl.ds(sid * NSLICE, NSLICE)])
        plsc.subcore_barrier()

        def grp(g, _):
            off = base + g * chunk * nbuf
            cps = []
            for b in range(nbuf):
                o = off + b * chunk
                ci = pltpu.async_copy(dst_hbm.at[pl.ds(o, chunk)], idxs[b],
                                      sems[b])
                cr = pltpu.async_copy(h_hbm.at[pl.ds(o, chunk)], rows[b],
                                      ss)
                cps.append((ci, cr))
            for b in range(nbuf):
                cps[b][0].wait()
                cps[b][1].wait()
                pltpu.sync_copy(rows[b], sh.at[idxs[b]], add=True)
            return 0

        lax.fori_loop(0, ngrp, grp, 0)
        for t in range(tail):
            o = base + (ngrp * nbuf + t) * chunk
            pltpu.async_copy(dst_hbm.at[pl.ds(o, chunk)], idxs[t],
                             sems[t]).wait()
            pltpu.async_copy(h_hbm.at[pl.ds(o, chunk)], rows[t], ss).wait()
            pltpu.sync_copy(rows[t], sh.at[idxs[t]], add=True)
        plsc.subcore_barrier()
        pltpu.sync_copy(sh.at[pl.ds(sid * NSLICE, NSLICE)],
                        out_hbm.at[cid, pl.ds(sid * NSLICE, NSLICE)])

    return k(h, dst, zeros)[0]


_BLK = 8000


def _bn_from_sums(sref, qref):
    s = jnp.sum(sref[...], axis=0, keepdims=True)
    q = jnp.sum(qref[...], axis=0, keepdims=True)
    mu = s * (1.0 / N_EDGES)
    var = q * (1.0 / N_EDGES) - mu * mu
    a = jax.lax.rsqrt(var + EPS)
    return a, -mu * a


def _fused_linear(xs, combos, bias, relu, want_stats, block=_BLK):
    rows = xs[0].shape[0]
    fo = combos[0][1].shape[1]
    n = len(xs)
    nc = len(combos)
    affs = [aff for (_, _, aff) in combos if aff is not None]
    grid = (rows // block,)

    def body(*refs):
        xrefs = refs[:n]
        wrefs = refs[n:n + nc]
        bref = refs[n + nc]
        arefs = refs[n + nc + 1:n + nc + 1 + 2 * len(affs)]
        yref = refs[n + nc + 1 + 2 * len(affs)]
        vals = [xr[...] for xr in xrefs]
        acc = None
        ai = 0
        for (spec, _, aff), wr in zip(combos, wrefs):
            v = vals[spec] if isinstance(spec, int) else \
                vals[spec[0]] - vals[spec[1]]
            if aff is not None:
                a8, c8 = _bn_from_sums(arefs[2 * ai], arefs[2 * ai + 1])
                ai += 1
                v = v * a8 + c8
            t = jnp.dot(v, wr[...], preferred_element_type=jnp.float32)
            acc = t if acc is None else acc + t
        acc = acc + bref[0:1, :]
        if relu:
            acc = jnp.maximum(acc, 0.0)
        yref[...] = acc
        if want_stats:
            k0 = n + nc + 2 + 2 * len(affs)
            sref, qref = refs[k0:k0 + 2]

            @pl.when(pl.program_id(0) == 0)
            def _():
                sref[...] = jnp.zeros_like(sref)
                qref[...] = jnp.zeros_like(qref)

            sref[...] += acc.reshape(block // 8, 8, fo).sum(0)
            qref[...] += (acc * acc).reshape(block // 8, 8, fo).sum(0)

    in_specs = (
        [pl.BlockSpec((block, x.shape[1]), lambda i: (i, 0)) for x in xs]
        + [pl.BlockSpec(w.shape, lambda i: (0, 0)) for (_, w, _) in combos]
        + [pl.BlockSpec((8, fo), lambda i: (0, 0))]
    )
    for (s8, q8) in affs:
        in_specs += [pl.BlockSpec(s8.shape, lambda i: (0, 0))] * 2
    out_shape = [jax.ShapeDtypeStruct((rows, fo), jnp.float32)]
    out_specs = [pl.BlockSpec((block, fo), lambda i: (i, 0))]
    if want_stats:
        out_shape += [jax.ShapeDtypeStruct((8, fo), jnp.float32)] * 2
        out_specs += [pl.BlockSpec((8, fo), lambda i: (0, 0))] * 2
    bias8 = jnp.broadcast_to(bias[None, :], (8, fo))
    aflat = [r for (s8, q8) in affs for r in (s8, q8)]
    res = pl.pallas_call(body, grid=grid, in_specs=in_specs,
                         out_specs=out_specs, out_shape=out_shape)(
                             *xs, *[w for (_, w, _) in combos], bias8, *aflat)
    if want_stats:
        return res[0], res[1], res[2]
    return res[0]


def _head_mlp(x, ws, bs, relus, rows, aff=None, block=_BLK):
    nl = len(ws)
    grid = (rows // block,)
    fo = ws[-1].shape[1]

    def body(xref, *refs):
        wrefs = refs[:nl]
        brefs = refs[nl:2 * nl]
        if aff is not None:
            s8, q8 = refs[2 * nl:2 * nl + 2]
            yref = refs[2 * nl + 2]
        else:
            yref = refs[2 * nl]
        h = xref[...]
        if aff is not None:
            a8, c8 = _bn_from_sums(s8, q8)
            h = h * a8 + c8
        for k in range(nl):
            h = jnp.dot(h, wrefs[k][...], preferred_element_type=jnp.float32)
            h = h + brefs[k][0:1, :]
            if relus[k]:
                h = jnp.maximum(h, 0.0)
        yref[...] = h

    in_specs = (
        [pl.BlockSpec((block, x.shape[1]), lambda i: (i, 0))]
        + [pl.BlockSpec(w.shape, lambda i: (0, 0)) for w in ws]
        + [pl.BlockSpec((8, w.shape[1]), lambda i: (0, 0)) for w in ws]
    )
    b8 = [jnp.broadcast_to(b[None, :], (8, b.shape[0])) for b in bs]
    args = [x, *ws, *b8]
    if aff is not None:
        s8, q8 = aff
        in_specs += [pl.BlockSpec(s8.shape, lambda i: (0, 0))] * 2
        args += [s8, q8]
    return pl.pallas_call(
        body, grid=grid, in_specs=in_specs,
        out_specs=pl.BlockSpec((block, fo), lambda i: (i, 0)),
        out_shape=jax.ShapeDtypeStruct((rows, fo), jnp.float32),
    )(*args)


def _stats_only(xs, cols_spec, block=_BLK):
    rows = xs[0].shape[0]
    n = len(xs)
    grid = (rows // block,)

    def body(*refs):
        xrefs = refs[:n]
        orefs = refs[n:]

        @pl.when(pl.program_id(0) == 0)
        def _():
            for r in orefs:
                r[...] = jnp.zeros_like(r)

        vals = [xr[...] for xr in xrefs]
        cols = [vals[c] if isinstance(c, int) else vals[c[0]] - vals[c[1]]
                for c in cols_spec]
        for k, v in enumerate(cols):
            f = v.shape[1]
            orefs[2 * k][...] += v.reshape(block // 8, 8, f).sum(0)
            orefs[2 * k + 1][...] += (v * v).reshape(block // 8, 8, f).sum(0)

    in_specs = [pl.BlockSpec((block, x.shape[1]), lambda i: (i, 0)) for x in xs]
    feats = [xs[c].shape[1] if isinstance(c, int) else xs[c[0]].shape[1]
             for c in cols_spec]
    out_shape = []
    out_specs = []
    for f in feats:
        out_shape += [jax.ShapeDtypeStruct((8, f), jnp.float32)] * 2
        out_specs += [pl.BlockSpec((8, f), lambda i: (0, 0))] * 2
    res = pl.pallas_call(body, grid=grid, in_specs=in_specs,
                         out_specs=out_specs, out_shape=out_shape)(*xs)
    return list(res)


_NBLK = 2048


def _node_affine(P, indeg_p, outdeg_p, s8, q8, feat):
    grid = (NP // _NBLK,)

    def body(pref, iref, oref, sref, qref, xref, ss, qs, sd, qd):
        a8, c8 = _bn_from_sums(sref, qref)
        s = pref[0] + pref[1]
        ind = iref[0, :, 0:1] + iref[1, :, 0:1]
        outd = oref[0, :, 0:1] + oref[1, :, 0:1]
        x = a8 * s + c8 * ind
        xref[...] = x

        @pl.when(pl.program_id(0) == 0)
        def _():
            for r in (ss, qs, sd, qd):
                r[...] = jnp.zeros_like(r)

        xx = x * x
        ss[...] += (x * outd).reshape(_NBLK // 8, 8, feat).sum(0)
        qs[...] += (xx * outd).reshape(_NBLK // 8, 8, feat).sum(0)
        sd[...] += (x * ind).reshape(_NBLK // 8, 8, feat).sum(0)
        qd[...] += (xx * ind).reshape(_NBLK // 8, 8, feat).sum(0)

    in_specs = [
        pl.BlockSpec((2, _NBLK, feat), lambda i: (0, i, 0)),
        pl.BlockSpec((2, _NBLK, 16), lambda i: (0, i, 0)),
        pl.BlockSpec((2, _NBLK, 16), lambda i: (0, i, 0)),
        pl.BlockSpec((8, feat), lambda i: (0, 0)),
        pl.BlockSpec((8, feat), lambda i: (0, 0)),
    ]
    out_shape = [jax.ShapeDtypeStruct((NP, feat), jnp.float32)] + \
        [jax.ShapeDtypeStruct((8, feat), jnp.float32)] * 4
    out_specs = [pl.BlockSpec((_NBLK, feat), lambda i: (i, 0))] + \
        [pl.BlockSpec((8, feat), lambda i: (0, 0))] * 4
    res = pl.pallas_call(body, grid=grid, in_specs=in_specs,
                         out_specs=out_specs, out_shape=out_shape)(
                             P, indeg_p, outdeg_p, s8, q8)
    return res[0], res[1:]


def kernel(node_feats, edge_feats, params, edge_index):
    p = params
    E = N_EDGES
    src = edge_index[0]
    dst = edge_index[1]
    ones16 = jnp.ones((CHUNK, 16), jnp.float32)
    zeros16 = jnp.zeros((NP, 16), jnp.float32)

    g1d, g1s, indeg_p, outdeg_p = _sc_gather16_deg(
        node_feats, src, dst, ones16, zeros16)

    aff_ef = _stats_only([edge_feats], [0])
    (s_a, q_a, s_b, q_b) = _stats_only([g1d, g1s], [0, (1, 0)])
    w0t = p["nmm1_w0"].T
    h, s, q = _fused_linear(
        [g1d, g1s],
        [(0, w0t[:16], (s_a, q_a)), ((1, 0), w0t[16:], (s_b, q_b))],
        p["nmm1_b0"], True, True)
    h, s, q = _fused_linear(
        [h], [(0, p["nmm1_w1"].T, (s, q))], p["nmm1_b1"], True, True)
    h13, s, q = _fused_linear(
        [h], [(0, p["nmm1_w2"].T, (s, q))], p["nmm1_b2"], True, True)

    zeros64 = jnp.zeros((NP, 64), jnp.float32)
    S1 = _sc_scatter_add(h13, dst, zeros64, 64)
    x1, (ss1, qs1, sd1, qd1) = _node_affine(S1, indeg_p, outdeg_p, s, q, 64)

    g2d, g2s = _sc_gather2(x1, src, dst, 64)

    w0t3 = p["nmm2_w0"].T
    h, s, q = _fused_linear(
        [g2d, g2s],
        [(0, w0t3[:64], None), ((1, 0), w0t3[64:], None)],
        p["nmm2_b0"], True, True)
    h, s, q = _fused_linear(
        [h], [(0, p["nmm2_w1"].T, (s, q))], p["nmm2_b1"], True, True)
    h33, s3, q3 = _fused_linear(
        [h], [(0, p["nmm2_w2"].T, (s, q))], p["nmm2_b2"], True, True)

    zeros128 = jnp.zeros((NP, 128), jnp.float32)
    S2 = _sc_scatter_add(h33, dst, zeros128, 128)

    v0t = p["emm1_w0"].T
    h, s, q = _fused_linear(
        [edge_feats, g2s, g2d],
        [(0, v0t[:19], aff_ef), (1, v0t[19:83], (ss1, qs1)),
         (2, v0t[83:147], (sd1, qd1))],
        p["emm1_b0"], True, True)
    h, s, q = _fused_linear(
        [h], [(0, p["emm1_w1"].T, (s, q))], p["emm1_b1"], True, True)
    e1raw, s_e1, q_e1 = _fused_linear(
        [h], [(0, p["emm1_w2"].T, (s, q))], p["emm1_b2"], True, True)

    x2, (ss2, qs2, sd2, qd2) = _node_affine(S2, indeg_p, outdeg_p, s3, q3, 128)

    g3d, g3s = _sc_gather2(x2, src, dst, 128)
    u0t = p["emm2_w0"].T
    h, s, q = _fused_linear(
        [e1raw, g3s, g3d],
        [(0, u0t[:76], (s_e1, q_e1)), (1, u0t[76:204], (ss2, qs2)),
         (2, u0t[204:332], (sd2, qd2))],
        p["emm2_b0"], True, True)
    h, s, q = _fused_linear(
        [h], [(0, p["emm2_w1"].T, (s, q))], p["emm2_b1"], True, True)
    e2raw, s, q = _fused_linear(
        [h], [(0, p["emm2_w2"].T, (s, q))], p["emm2_b2"], True, True)
    aff_e2 = (s, q)

    nout = _head_mlp(
        x2,
        [p["nhead_w0"].T, p["nhead_w1"].T, p["nhead_w2"].T, p["nhead_w3"].T],
        [p["nhead_b0"], p["nhead_b1"], p["nhead_b2"], p["nhead_b3"]],
        [True, True, False, False], N_NODES, block=2000)

    eout = _head_mlp(
        e2raw,
        [p["ehead_w0"].T, p["ehead_w1"].T, p["ehead_w2"].T, p["ehead_w3"].T,
         p["ehead_w4"].T],
        [p["ehead_b0"], p["ehead_b1"], p["ehead_b2"], p["ehead_b3"],
         p["ehead_b4"]],
        [False, True, True, False, False], E, aff=aff_e2)

    return (nout, eout)
